# Initial kernel scaffold; baseline (speedup 1.0000x reference)
import jax
import jax.numpy as jnp
from jax.experimental import pallas as pl


def kernel(x, B, H, W, embedding_weight):
    raise NotImplementedError("write your pallas kernel here")


# speedup vs baseline: 9.3015x; 9.3015x over previous
import functools

import jax
import jax.numpy as jnp
from jax import lax
from jax.experimental import pallas as pl
from jax.experimental.pallas import tpu as pltpu
from jax.experimental.pallas import tpu_sc as plsc

_B, _H, _W = 8, 32, 32
_BETA = 1.0

_TN = 512


def _argmin_body(x2_ref, e2_ref, x_ref, et_ref, idx_ref):
    mm = jnp.dot(x_ref[...], et_ref[...], preferred_element_type=jnp.float32)
    d = (x2_ref[...] + e2_ref[...]) - 2.0 * mm
    m = jnp.min(d, axis=1, keepdims=True)
    ii = lax.broadcasted_iota(jnp.int32, d.shape, 1)
    idx_ref[...] = jnp.min(jnp.where(d == m, ii, d.shape[1]), axis=1)


def _argmin_call(x2, e2, x, et, *, interpret=False):
    n, dd = x.shape
    k = et.shape[1]
    grid = (n // _TN,)
    return pl.pallas_call(
        _argmin_body,
        grid=grid,
        in_specs=[
            pl.BlockSpec((_TN, 1), lambda i: (i, 0)),
            pl.BlockSpec((1, k), lambda i: (0, 0)),
            pl.BlockSpec((_TN, dd), lambda i: (i, 0)),
            pl.BlockSpec((dd, k), lambda i: (0, 0)),
        ],
        out_specs=pl.BlockSpec((_TN,), lambda i: (i,)),
        out_shape=jax.ShapeDtypeStruct((n,), jnp.int32),
        interpret=interpret,
    )(x2, e2, x, et)


_NC, _NS = 2, 16
_NW = _NC * _NS
_ROWS_PER_W = 256
_CH = 128


def _sc_gather_body(table_hbm, idx_hbm, out_hbm, idx_v, rows_v, sem):
    wid = lax.axis_index("s") * _NC + lax.axis_index("c")
    pltpu.sync_copy(idx_hbm.at[pl.ds(wid * 2, 2)], idx_v)
    cp0 = pltpu.async_copy(table_hbm.at[idx_v.at[0]], rows_v.at[pl.ds(0, _CH)], sem)
    cp1 = pltpu.async_copy(table_hbm.at[idx_v.at[1]], rows_v.at[pl.ds(_CH, _CH)], sem)
    cp0.wait()
    cp1.wait()
    pltpu.sync_copy(rows_v, out_hbm.at[pl.ds(wid * _ROWS_PER_W, _ROWS_PER_W)])


def _sc_gather(table, idx2d):
    k, d = table.shape
    n = idx2d.shape[0] * idx2d.shape[1]
    kern = pl.kernel(
        _sc_gather_body,
        out_type=jax.ShapeDtypeStruct((n, d), jnp.float32),
        mesh=plsc.VectorSubcoreMesh(core_axis_name="c", subcore_axis_name="s"),
        scratch_types=[
            pltpu.VMEM((2, _CH), jnp.int32),
            pltpu.VMEM((_ROWS_PER_W, d), jnp.float32),
            pltpu.SemaphoreType.DMA,
        ],
    )
    return kern(table, idx2d)


def _trans_body(q_ref, x_ref, qt_ref, lp_ref):
    qb = q_ref[0]
    diff = qb - x_ref[0]
    lp_ref[...] = jnp.full((1, 1, 128), jnp.sum(diff * diff), jnp.float32)
    qt_ref[0] = qb.T


def _trans_call(q3, x3, *, interpret=False):
    b, hw, d = q3.shape
    return pl.pallas_call(
        _trans_body,
        grid=(b,),
        in_specs=[
            pl.BlockSpec((1, hw, d), lambda i: (i, 0, 0)),
            pl.BlockSpec((1, hw, d), lambda i: (i, 0, 0)),
        ],
        out_specs=[
            pl.BlockSpec((1, d, hw), lambda i: (i, 0, 0)),
            pl.BlockSpec((1, 1, 128), lambda i: (i, 0, 0)),
        ],
        out_shape=[
            jax.ShapeDtypeStruct((b, d, hw), jnp.float32),
            jax.ShapeDtypeStruct((b, 1, 128), jnp.float32),
        ],
        interpret=interpret,
    )(q3, x3)


def kernel(x, B, H, W, embedding_weight):
    n, d = x.shape
    k = embedding_weight.shape[0]
    x2 = jnp.sum(x ** 2, axis=1, keepdims=True)
    e2 = jnp.sum(embedding_weight ** 2, axis=1)[None, :]
    et = embedding_weight.T

    idx = _argmin_call(x2, e2, x, et)
    q = _sc_gather(embedding_weight, idx.reshape(n // _CH, _CH))

    hw = _H * _W
    qt, lp = _trans_call(q.reshape(_B, hw, d), x.reshape(_B, hw, d))
    loss = 2.0 * jnp.sum(lp[:, 0, 0]) / (n * d)
    quantized = qt.reshape(_B, d, _H, _W)
    return (loss, quantized, idx[:, None])


# speedup vs baseline: 9.4952x; 1.0208x over previous
import functools

import jax
import jax.numpy as jnp
from jax import lax
from jax.experimental import pallas as pl
from jax.experimental.pallas import tpu as pltpu
from jax.experimental.pallas import tpu_sc as plsc

_B, _H, _W = 8, 32, 32
_BETA = 1.0

_TN = 1024
_TK = 1024


def _argmin_body(x2_ref, e2_ref, x_ref, e_ref, idx_ref):
    xb = x_ref[...]
    x2 = x2_ref[...]
    nchunk = e_ref.shape[0] // _TK
    run_min = None
    run_c = None
    for c in range(nchunk):
        ec = e_ref[pl.ds(c * _TK, _TK), :]
        mm = lax.dot_general(xb, ec, (((1,), (1,)), ((), ())),
                             preferred_element_type=jnp.float32)
        d = (x2 + e2_ref[:, pl.ds(c * _TK, _TK)]) - 2.0 * mm
        if c == 0:
            run_min, run_c = d, jnp.zeros(d.shape, jnp.int32)
        else:
            lt = d < run_min
            run_min = jnp.minimum(run_min, d)
            run_c = jnp.where(lt, c, run_c)
    m = jnp.min(run_min, axis=1, keepdims=True)
    jj = lax.broadcasted_iota(jnp.int32, run_min.shape, 1)
    kfull = run_c * _TK + jj
    big = nchunk * _TK
    idx_ref[...] = jnp.min(jnp.where(run_min == m, kfull, big), axis=1)


def _argmin_call(x2, e2, x, e, *, interpret=False):
    n, dd = x.shape
    k = e.shape[0]
    grid = (n // _TN,)
    return pl.pallas_call(
        _argmin_body,
        grid=grid,
        in_specs=[
            pl.BlockSpec((_TN, 1), lambda i: (i, 0)),
            pl.BlockSpec((1, k), lambda i: (0, 0)),
            pl.BlockSpec((_TN, dd), lambda i: (i, 0)),
            pl.BlockSpec((k, dd), lambda i: (0, 0)),
        ],
        out_specs=pl.BlockSpec((_TN,), lambda i: (i,)),
        out_shape=jax.ShapeDtypeStruct((n,), jnp.int32),
        interpret=interpret,
    )(x2, e2, x, e)


_NC, _NS = 2, 16
_NW = _NC * _NS
_ROWS_PER_W = 256
_CH = 128


def _sc_gather_body(table_hbm, idx_hbm, out_hbm, idx_v, rows_v, sem):
    wid = lax.axis_index("s") * _NC + lax.axis_index("c")
    pltpu.sync_copy(idx_hbm.at[pl.ds(wid * 2, 2)], idx_v)
    cp0 = pltpu.async_copy(table_hbm.at[idx_v.at[0]], rows_v.at[pl.ds(0, _CH)], sem)
    cp1 = pltpu.async_copy(table_hbm.at[idx_v.at[1]], rows_v.at[pl.ds(_CH, _CH)], sem)
    cp0.wait()
    cp1.wait()
    pltpu.sync_copy(rows_v, out_hbm.at[pl.ds(wid * _ROWS_PER_W, _ROWS_PER_W)])


def _sc_gather(table, idx2d):
    k, d = table.shape
    n = idx2d.shape[0] * idx2d.shape[1]
    kern = pl.kernel(
        _sc_gather_body,
        out_type=jax.ShapeDtypeStruct((n, d), jnp.float32),
        mesh=plsc.VectorSubcoreMesh(core_axis_name="c", subcore_axis_name="s"),
        scratch_types=[
            pltpu.VMEM((2, _CH), jnp.int32),
            pltpu.VMEM((_ROWS_PER_W, d), jnp.float32),
            pltpu.SemaphoreType.DMA,
        ],
    )
    return kern(table, idx2d)


def _trans_body(q_ref, x_ref, qt_ref, lp_ref):
    qb = q_ref[0]
    diff = qb - x_ref[0]
    lp_ref[...] = jnp.full((1, 1, 128), jnp.sum(diff * diff), jnp.float32)
    qt_ref[0] = qb.T


def _trans_call(q3, x3, *, interpret=False):
    b, hw, d = q3.shape
    return pl.pallas_call(
        _trans_body,
        grid=(b,),
        in_specs=[
            pl.BlockSpec((1, hw, d), lambda i: (i, 0, 0)),
            pl.BlockSpec((1, hw, d), lambda i: (i, 0, 0)),
        ],
        out_specs=[
            pl.BlockSpec((1, d, hw), lambda i: (i, 0, 0)),
            pl.BlockSpec((1, 1, 128), lambda i: (i, 0, 0)),
        ],
        out_shape=[
            jax.ShapeDtypeStruct((b, d, hw), jnp.float32),
            jax.ShapeDtypeStruct((b, 1, 128), jnp.float32),
        ],
        interpret=interpret,
    )(q3, x3)


def kernel(x, B, H, W, embedding_weight):
    n, d = x.shape
    k = embedding_weight.shape[0]
    x2 = jnp.sum(x ** 2, axis=1, keepdims=True)
    e2 = jnp.sum(embedding_weight ** 2, axis=1)[None, :]

    idx = _argmin_call(x2, e2, x, embedding_weight)
    q = _sc_gather(embedding_weight, idx.reshape(n // _CH, _CH))

    hw = _H * _W
    qt, lp = _trans_call(q.reshape(_B, hw, d), x.reshape(_B, hw, d))
    loss = 2.0 * jnp.sum(lp[:, 0, 0]) / (n * d)
    quantized = qt.reshape(_B, d, _H, _W)
    return (loss, quantized, idx[:, None])
